# baseline (speedup 1.0000x reference)
import functools
import jax
import jax.numpy as jnp
from jax import lax
from jax.experimental import pallas as pl
from jax.experimental.pallas import tpu as pltpu, tpu_sc as plsc

GRID = 32
D_MODEL = 1024
HALF = D_MODEL // 2
B_TC = 40


def _tc_body(row_ref, col_ref, out_ref):
    col = col_ref[...]
    row = row_ref[...]
    first = jnp.broadcast_to(col[None, :, :], (GRID, GRID, HALF))
    second = jnp.broadcast_to(row[:, None, :], (GRID, GRID, HALF))
    out_ref[0] = jnp.concatenate([first, second], axis=-1)


def _sc_body(n_batch, row_hbm, col_hbm, out_hbm, chunk, sem):
    wid = lax.axis_index("s") * 2 + lax.axis_index("c")
    pltpu.sync_copy(col_hbm, chunk.at[:, pl.ds(0, HALF)])
    for w in range(GRID):
        pltpu.sync_copy(row_hbm.at[wid], chunk.at[w, pl.ds(HALF, HALF)])
    for g in range(0, n_batch, 8):
        copies = [
            pltpu.async_copy(chunk, out_hbm.at[b, pl.ds(wid * GRID, GRID), :], sem)
            for b in range(g, min(g + 8, n_batch))
        ]
        for c in copies:
            c.wait()


def kernel(x, row_embed, col_embed):
    b = x.shape[0]
    b_tc = min(B_TC, b)
    b_sc = b - b_tc

    tc_out = pl.pallas_call(
        _tc_body,
        grid=(b_tc,),
        in_specs=[
            pl.BlockSpec((GRID, HALF), lambda i: (0, 0)),
            pl.BlockSpec((GRID, HALF), lambda i: (0, 0)),
        ],
        out_specs=pl.BlockSpec((1, GRID, GRID, D_MODEL), lambda i: (i, 0, 0, 0)),
        out_shape=jax.ShapeDtypeStruct((b_tc, GRID, GRID, D_MODEL), jnp.float32),
    )(row_embed, col_embed).reshape(b_tc, GRID * GRID, D_MODEL)

    if b_sc == 0:
        return tc_out

    mesh = plsc.VectorSubcoreMesh(core_axis_name="c", subcore_axis_name="s")
    sc_out = functools.partial(
        pl.kernel,
        out_type=jax.ShapeDtypeStruct((b_sc, GRID * GRID, D_MODEL), jnp.float32),
        mesh=mesh,
        scratch_types=[
            pltpu.VMEM((GRID, D_MODEL), jnp.float32),
            pltpu.SemaphoreType.DMA,
        ],
    )(functools.partial(_sc_body, b_sc))(row_embed, col_embed)

    return jnp.concatenate([tc_out, sc_out], axis=0)


# speedup vs baseline: 1.9130x; 1.9130x over previous
import functools
import jax
import jax.numpy as jnp
from jax import lax
from jax.experimental import pallas as pl
from jax.experimental.pallas import tpu as pltpu, tpu_sc as plsc

GRID = 32
D_MODEL = 1024
HALF = D_MODEL // 2
N_HCHUNK = 16
N_BGROUP = 2
FIRE = 16


def _sc_body(n_batch, row_hbm, col_hbm, out_hbm, chunk, sem):
    wid = lax.axis_index("s") * 2 + lax.axis_index("c")
    hc = wid % N_HCHUNK
    bg = wid // N_HCHUNK
    nb = n_batch // N_BGROUP
    for sub in range(2):
        pltpu.sync_copy(col_hbm, chunk.at[pl.ds(sub * GRID, GRID), pl.ds(0, HALF)])
        for w in range(GRID):
            pltpu.sync_copy(
                row_hbm.at[hc * 2 + sub],
                chunk.at[sub * GRID + w, pl.ds(HALF, HALF)],
            )
    for g in range(0, nb, FIRE):
        copies = [
            pltpu.async_copy(
                chunk,
                out_hbm.at[bg * nb + b, pl.ds(hc * 2 * GRID, 2 * GRID), :],
                sem,
            )
            for b in range(g, min(g + FIRE, nb))
        ]
        for c in copies:
            c.wait()


def kernel(x, row_embed, col_embed):
    b = x.shape[0]
    mesh = plsc.VectorSubcoreMesh(core_axis_name="c", subcore_axis_name="s")
    run = functools.partial(
        pl.kernel,
        out_type=jax.ShapeDtypeStruct((b, GRID * GRID, D_MODEL), jnp.float32),
        mesh=mesh,
        scratch_types=[
            pltpu.VMEM((2 * GRID, D_MODEL), jnp.float32),
            pltpu.SemaphoreType.DMA,
        ],
    )(functools.partial(_sc_body, b))
    return run(row_embed, col_embed)


# speedup vs baseline: 2.5124x; 1.3133x over previous
import functools
import jax
import jax.numpy as jnp
from jax import lax
from jax.experimental import pallas as pl
from jax.experimental.pallas import tpu as pltpu, tpu_sc as plsc

GRID = 32
D_MODEL = 1024
HALF = D_MODEL // 2
FIRE = 8


def _sc_body(n_batch, row_hbm, col_hbm, out_hbm, chunk, sem):
    wid = lax.axis_index("s") * 2 + lax.axis_index("c")
    stage = [pltpu.async_copy(col_hbm, chunk.at[:, pl.ds(0, HALF)], sem)]
    stage += [
        pltpu.async_copy(row_hbm.at[wid], chunk.at[w, pl.ds(HALF, HALF)], sem)
        for w in range(GRID)
    ]
    for c in stage:
        c.wait()
    for g in range(0, n_batch, FIRE):
        copies = [
            pltpu.async_copy(chunk, out_hbm.at[b, pl.ds(wid * GRID, GRID), :], sem)
            for b in range(g, min(g + FIRE, n_batch))
        ]
        for c in copies:
            c.wait()


def kernel(x, row_embed, col_embed):
    b = x.shape[0]
    mesh = plsc.VectorSubcoreMesh(core_axis_name="c", subcore_axis_name="s")
    run = functools.partial(
        pl.kernel,
        out_type=jax.ShapeDtypeStruct((b, GRID * GRID, D_MODEL), jnp.float32),
        mesh=mesh,
        scratch_types=[
            pltpu.VMEM((GRID, D_MODEL), jnp.float32),
            pltpu.SemaphoreType.DMA,
        ],
    )(functools.partial(_sc_body, b))
    return run(row_embed, col_embed)
